# baseline (speedup 1.0000x reference)
import functools

import jax
import jax.numpy as jnp
from jax import lax
from jax.experimental import pallas as pl
from jax.experimental.pallas import tpu as pltpu
from jax.experimental.pallas import tpu_sc as plsc

EMB = 64
BATCH = 16384
VOCAB = 1000000
WIDE = 2 * EMB
SPLIT = 499968
_LB = 3968
_NBLK = 126
_GRID = _NBLK + 1
_WROWS = _GRID * _LB

_INFO = plsc.get_sparse_core_info()
_NC = _INFO.num_cores
_NS = _INFO.num_subcores
_NW = _NC * _NS
_BPW = BATCH // _NW
_CHUNK = 128
_NCHUNK = _BPW // _CHUNK
_L = 16


def _widen_block(lo_ref, hi_ref, o_ref):
    ii = lax.broadcasted_iota(jnp.int32, (EMB, EMB), 0)
    jj = lax.broadcasted_iota(jnp.int32, (EMB, EMB), 1)
    eye = (ii == jj).astype(jnp.float32)
    o_ref[:, 0:EMB] = lax.dot_general(
        lo_ref[...], eye, (((0,), (0,)), ((), ())),
        preferred_element_type=jnp.float32)
    o_ref[:, EMB:WIDE] = lax.dot_general(
        hi_ref[...], eye, (((0,), (0,)), ((), ())),
        preferred_element_type=jnp.float32)


def _tc_widen(tableT):
    lo_map = lambda k: (0, jnp.where(k < _NBLK, k, 2 * _NBLK))
    hi_map = lambda k: (0, jnp.where(k < _NBLK, _NBLK + k, 2 * _NBLK))
    return pl.pallas_call(
        _widen_block,
        grid=(_GRID,),
        in_specs=[
            pl.BlockSpec((EMB, _LB), lo_map),
            pl.BlockSpec((EMB, _LB), hi_map),
        ],
        out_specs=pl.BlockSpec((_LB, WIDE), lambda k: (k, 0)),
        out_shape=jax.ShapeDtypeStruct((_WROWS, WIDE), jnp.float32),
    )(tableT, tableT)


def _sc_gather(wide, idx3d):
    mesh = plsc.VectorSubcoreMesh(core_axis_name="c", subcore_axis_name="s")

    @functools.partial(
        pl.kernel,
        out_type=jax.ShapeDtypeStruct((BATCH, WIDE), jnp.float32),
        mesh=mesh,
        scratch_types=[
            pltpu.VMEM((_NCHUNK, _CHUNK), jnp.int32),
            pltpu.VMEM((_NCHUNK, _CHUNK), jnp.int32),
            pltpu.VMEM((_BPW, WIDE), jnp.float32),
            pltpu.SemaphoreType.DMA,
        ],
    )
    def gather_kernel(wide_hbm, idx_hbm, out_hbm, idx_v, idxr_v, rows_v, sem):
        wid = lax.axis_index("s") * _NC + lax.axis_index("c")
        base = wid * _BPW
        pltpu.sync_copy(idx_hbm.at[wid], idx_v)
        for j in range(_NCHUNK):
            for k in range(_CHUNK // _L):
                sl = pl.ds(k * _L, _L)
                v = idx_v[j, sl]
                idxr_v[j, sl] = v - jnp.where(v >= SPLIT, SPLIT, 0)
        copies = [
            pltpu.async_copy(
                wide_hbm.at[idxr_v.at[j]],
                rows_v.at[pl.ds(j * _CHUNK, _CHUNK)],
                sem,
            )
            for j in range(_NCHUNK)
        ]
        for c in copies:
            c.wait()
        pltpu.sync_copy(rows_v, out_hbm.at[pl.ds(base, _BPW)])

    return gather_kernel(wide, idx3d)


_BLK = 2048


def _mlp_block(xw_ref, idx_ref, w1_ref, b1_ref, w2_ref, b2_ref, o_ref):
    xw = xw_ref[...]
    idx = idx_ref[...]
    hi = jnp.logical_and(idx >= SPLIT, idx < 2 * SPLIT)
    x = jnp.where(hi, xw[:, EMB:], xw[:, :EMB])
    h = jnp.dot(x, w1_ref[...], preferred_element_type=jnp.float32)
    h = h * 8.0 + b1_ref[...]
    h = jnp.where(h > 0.0, h, jnp.exp(h) - 1.0)
    o_ref[...] = jnp.dot(h, w2_ref[...], preferred_element_type=jnp.float32) + b2_ref[...]


def _tc_mlp(xw, idx2d, W1, b1, W2, b2):
    grid = (BATCH // _BLK,)
    return pl.pallas_call(
        _mlp_block,
        grid=grid,
        in_specs=[
            pl.BlockSpec((_BLK, WIDE), lambda i: (i, 0)),
            pl.BlockSpec((_BLK, 1), lambda i: (i, 0)),
            pl.BlockSpec((EMB, EMB), lambda i: (0, 0)),
            pl.BlockSpec((1, EMB), lambda i: (0, 0)),
            pl.BlockSpec((EMB, EMB), lambda i: (0, 0)),
            pl.BlockSpec((1, EMB), lambda i: (0, 0)),
        ],
        out_specs=pl.BlockSpec((_BLK, EMB), lambda i: (i, 0)),
        out_shape=jax.ShapeDtypeStruct((BATCH, EMB), jnp.float32),
    )(xw, idx2d, W1, b1, W2, b2)


def kernel(indices, table, W1, b1, W2, b2):
    idx = indices.astype(jnp.int32)
    wide = _tc_widen(table.T)
    xw = _sc_gather(wide, idx.reshape(_NW, _NCHUNK, _CHUNK))
    return _tc_mlp(xw, idx.reshape(BATCH, 1), W1, b1.reshape(1, EMB),
                   W2, b2.reshape(1, EMB))


# speedup vs baseline: 1.1189x; 1.1189x over previous
import functools

import jax
import jax.numpy as jnp
from jax import lax
from jax.experimental import pallas as pl
from jax.experimental.pallas import tpu as pltpu
from jax.experimental.pallas import tpu_sc as plsc

EMB = 64
BATCH = 16384
VOCAB = 1000000
WIDE = 2 * EMB
SPLIT = 499968
_LB = 7936
_NBLK = 63
_GRID = _NBLK + 1
_WROWS = _GRID * _LB

_INFO = plsc.get_sparse_core_info()
_NC = _INFO.num_cores
_NS = _INFO.num_subcores
_NW = _NC * _NS
_BPW = BATCH // _NW
_CHUNK = 128
_NCHUNK = _BPW // _CHUNK
_L = 16


def _widen_block(lo_ref, hi_ref, o_ref):
    ii = lax.broadcasted_iota(jnp.int32, (EMB, EMB), 0)
    jj = lax.broadcasted_iota(jnp.int32, (EMB, EMB), 1)
    eye = (ii == jj).astype(jnp.float32)
    o_ref[:, 0:EMB] = lax.dot_general(
        lo_ref[...], eye, (((0,), (0,)), ((), ())),
        preferred_element_type=jnp.float32)
    o_ref[:, EMB:WIDE] = lax.dot_general(
        hi_ref[...], eye, (((0,), (0,)), ((), ())),
        preferred_element_type=jnp.float32)


def _tc_widen(tableT):
    lo_map = lambda k: (0, jnp.where(k < _NBLK, k, 2 * _NBLK))
    hi_map = lambda k: (0, jnp.where(k < _NBLK, _NBLK + k, 2 * _NBLK))
    return pl.pallas_call(
        _widen_block,
        grid=(_GRID,),
        in_specs=[
            pl.BlockSpec((EMB, _LB), lo_map),
            pl.BlockSpec((EMB, _LB), hi_map),
        ],
        out_specs=pl.BlockSpec((_LB, WIDE), lambda k: (k, 0)),
        out_shape=jax.ShapeDtypeStruct((_WROWS, WIDE), jnp.float32),
    )(tableT, tableT)


def _sc_gather(wide, idx3d):
    mesh = plsc.VectorSubcoreMesh(core_axis_name="c", subcore_axis_name="s")

    @functools.partial(
        pl.kernel,
        out_type=jax.ShapeDtypeStruct((BATCH, WIDE), jnp.float32),
        mesh=mesh,
        scratch_types=[
            pltpu.VMEM((_NCHUNK, _CHUNK), jnp.int32),
            pltpu.VMEM((_NCHUNK, _CHUNK), jnp.int32),
            pltpu.VMEM((_BPW, WIDE), jnp.float32),
            pltpu.SemaphoreType.DMA,
        ],
    )
    def gather_kernel(wide_hbm, idx_hbm, out_hbm, idx_v, idxr_v, rows_v, sem):
        wid = lax.axis_index("s") * _NC + lax.axis_index("c")
        base = wid * _BPW
        pltpu.sync_copy(idx_hbm.at[wid], idx_v)
        for j in range(_NCHUNK):
            for k in range(_CHUNK // _L):
                sl = pl.ds(k * _L, _L)
                v = idx_v[j, sl]
                idxr_v[j, sl] = v - jnp.where(v >= SPLIT, SPLIT, 0)
        copies = [
            pltpu.async_copy(
                wide_hbm.at[idxr_v.at[j]],
                rows_v.at[pl.ds(j * _CHUNK, _CHUNK)],
                sem,
            )
            for j in range(_NCHUNK)
        ]
        for c in copies:
            c.wait()
        pltpu.sync_copy(rows_v, out_hbm.at[pl.ds(base, _BPW)])

    return gather_kernel(wide, idx3d)


_BLK = 2048


def _mlp_block(xw_ref, idx_ref, w1_ref, b1_ref, w2_ref, b2_ref, o_ref):
    xw = xw_ref[...]
    idx = idx_ref[...]
    hi = jnp.logical_and(idx >= SPLIT, idx < 2 * SPLIT)
    x = jnp.where(hi, xw[:, EMB:], xw[:, :EMB])
    h = jnp.dot(x, w1_ref[...], preferred_element_type=jnp.float32)
    h = h * 8.0 + b1_ref[...]
    h = jnp.where(h > 0.0, h, jnp.exp(h) - 1.0)
    o_ref[...] = jnp.dot(h, w2_ref[...], preferred_element_type=jnp.float32) + b2_ref[...]


def _tc_mlp(xw, idx2d, W1, b1, W2, b2):
    grid = (BATCH // _BLK,)
    return pl.pallas_call(
        _mlp_block,
        grid=grid,
        in_specs=[
            pl.BlockSpec((_BLK, WIDE), lambda i: (i, 0)),
            pl.BlockSpec((_BLK, 1), lambda i: (i, 0)),
            pl.BlockSpec((EMB, EMB), lambda i: (0, 0)),
            pl.BlockSpec((1, EMB), lambda i: (0, 0)),
            pl.BlockSpec((EMB, EMB), lambda i: (0, 0)),
            pl.BlockSpec((1, EMB), lambda i: (0, 0)),
        ],
        out_specs=pl.BlockSpec((_BLK, EMB), lambda i: (i, 0)),
        out_shape=jax.ShapeDtypeStruct((BATCH, EMB), jnp.float32),
    )(xw, idx2d, W1, b1, W2, b2)


def kernel(indices, table, W1, b1, W2, b2):
    idx = indices.astype(jnp.int32)
    wide = _tc_widen(table.T)
    xw = _sc_gather(wide, idx.reshape(_NW, _NCHUNK, _CHUNK))
    return _tc_mlp(xw, idx.reshape(BATCH, 1), W1, b1.reshape(1, EMB),
                   W2, b2.reshape(1, EMB))


# speedup vs baseline: 1.1751x; 1.0501x over previous
import functools

import jax
import jax.numpy as jnp
from jax import lax
from jax.experimental import pallas as pl
from jax.experimental.pallas import tpu as pltpu
from jax.experimental.pallas import tpu_sc as plsc

EMB = 64
BATCH = 16384
VOCAB = 1000000
WIDE = 2 * EMB
SPLIT = 499968
_LB = 16128
_NBLK = 31
_GRID = _NBLK + 1
_WROWS = _GRID * _LB

_INFO = plsc.get_sparse_core_info()
_NC = _INFO.num_cores
_NS = _INFO.num_subcores
_NW = _NC * _NS
_BPW = BATCH // _NW
_CHUNK = 128
_NCHUNK = _BPW // _CHUNK
_L = 16


def _widen_block(lo_ref, hi_ref, o_ref):
    ii = lax.broadcasted_iota(jnp.int32, (EMB, EMB), 0)
    jj = lax.broadcasted_iota(jnp.int32, (EMB, EMB), 1)
    eye = (ii == jj).astype(jnp.float32)
    o_ref[:, 0:EMB] = lax.dot_general(
        lo_ref[...], eye, (((0,), (0,)), ((), ())),
        preferred_element_type=jnp.float32)
    o_ref[:, EMB:WIDE] = lax.dot_general(
        hi_ref[...], eye, (((0,), (0,)), ((), ())),
        preferred_element_type=jnp.float32)


def _tc_widen(tableT):
    lo_map = lambda k: (0, jnp.where(k < _NBLK, k, 2 * _NBLK))
    hi_map = lambda k: (0, jnp.where(k < _NBLK, _NBLK + k, 2 * _NBLK))
    return pl.pallas_call(
        _widen_block,
        grid=(_GRID,),
        in_specs=[
            pl.BlockSpec((EMB, _LB), lo_map),
            pl.BlockSpec((EMB, _LB), hi_map),
        ],
        out_specs=pl.BlockSpec((_LB, WIDE), lambda k: (k, 0)),
        out_shape=jax.ShapeDtypeStruct((_WROWS, WIDE), jnp.float32),
    )(tableT, tableT)


def _sc_gather(wide, idx3d):
    mesh = plsc.VectorSubcoreMesh(core_axis_name="c", subcore_axis_name="s")

    @functools.partial(
        pl.kernel,
        out_type=jax.ShapeDtypeStruct((BATCH, WIDE), jnp.float32),
        mesh=mesh,
        scratch_types=[
            pltpu.VMEM((_NCHUNK, _CHUNK), jnp.int32),
            pltpu.VMEM((_NCHUNK, _CHUNK), jnp.int32),
            pltpu.VMEM((_BPW, WIDE), jnp.float32),
            pltpu.SemaphoreType.DMA,
        ],
    )
    def gather_kernel(wide_hbm, idx_hbm, out_hbm, idx_v, idxr_v, rows_v, sem):
        wid = lax.axis_index("s") * _NC + lax.axis_index("c")
        base = wid * _BPW
        pltpu.sync_copy(idx_hbm.at[wid], idx_v)
        for j in range(_NCHUNK):
            for k in range(_CHUNK // _L):
                sl = pl.ds(k * _L, _L)
                v = idx_v[j, sl]
                idxr_v[j, sl] = v - jnp.where(v >= SPLIT, SPLIT, 0)
        copies = [
            pltpu.async_copy(
                wide_hbm.at[idxr_v.at[j]],
                rows_v.at[pl.ds(j * _CHUNK, _CHUNK)],
                sem,
            )
            for j in range(_NCHUNK)
        ]
        for c in copies:
            c.wait()
        pltpu.sync_copy(rows_v, out_hbm.at[pl.ds(base, _BPW)])

    return gather_kernel(wide, idx3d)


_BLK = 2048


def _mlp_block(xw_ref, idx_ref, w1_ref, b1_ref, w2_ref, b2_ref, o_ref):
    xw = xw_ref[...]
    idx = idx_ref[...]
    hi = jnp.logical_and(idx >= SPLIT, idx < 2 * SPLIT)
    x = jnp.where(hi, xw[:, EMB:], xw[:, :EMB])
    h = jnp.dot(x, w1_ref[...], preferred_element_type=jnp.float32)
    h = h * 8.0 + b1_ref[...]
    h = jnp.where(h > 0.0, h, jnp.exp(h) - 1.0)
    o_ref[...] = jnp.dot(h, w2_ref[...], preferred_element_type=jnp.float32) + b2_ref[...]


def _tc_mlp(xw, idx2d, W1, b1, W2, b2):
    grid = (BATCH // _BLK,)
    return pl.pallas_call(
        _mlp_block,
        grid=grid,
        in_specs=[
            pl.BlockSpec((_BLK, WIDE), lambda i: (i, 0)),
            pl.BlockSpec((_BLK, 1), lambda i: (i, 0)),
            pl.BlockSpec((EMB, EMB), lambda i: (0, 0)),
            pl.BlockSpec((1, EMB), lambda i: (0, 0)),
            pl.BlockSpec((EMB, EMB), lambda i: (0, 0)),
            pl.BlockSpec((1, EMB), lambda i: (0, 0)),
        ],
        out_specs=pl.BlockSpec((_BLK, EMB), lambda i: (i, 0)),
        out_shape=jax.ShapeDtypeStruct((BATCH, EMB), jnp.float32),
    )(xw, idx2d, W1, b1, W2, b2)


def kernel(indices, table, W1, b1, W2, b2):
    idx = indices.astype(jnp.int32)
    wide = _tc_widen(table.T)
    xw = _sc_gather(wide, idx.reshape(_NW, _NCHUNK, _CHUNK))
    return _tc_mlp(xw, idx.reshape(BATCH, 1), W1, b1.reshape(1, EMB),
                   W2, b2.reshape(1, EMB))
